# Initial kernel scaffold; baseline (speedup 1.0000x reference)
import jax
import jax.numpy as jnp
from jax.experimental import pallas as pl


def kernel(x, emb_table):
    raise NotImplementedError("write your pallas kernel here")


# speedup vs baseline: 3.4879x; 3.4879x over previous
import functools
import math

import jax
import jax.numpy as jnp
from jax import lax
from jax.experimental import pallas as pl
from jax.experimental.pallas import tpu as pltpu
from jax.experimental.pallas import tpu_sc as plsc

D_MODEL = 64
SCALE = math.sqrt(D_MODEL)

NUM_CORES = 2
NUM_SUBCORES = 16
NUM_WORKERS = NUM_CORES * NUM_SUBCORES
CHUNK = 128
NBUF = 2


@functools.partial(jax.jit, static_argnums=(2,))
def _emb_lookup(idx3, table, n_chunks):
  mesh = plsc.VectorSubcoreMesh(core_axis_name="c", subcore_axis_name="s")
  n_rows = NUM_WORKERS * n_chunks * CHUNK

  scratch = [pltpu.VMEM((n_chunks, CHUNK), jnp.int32)]
  scratch += [pltpu.VMEM((CHUNK, D_MODEL), jnp.float32) for _ in range(NBUF)]
  scratch += [pltpu.SemaphoreType.DMA for _ in range(NBUF)]

  @functools.partial(
      pl.kernel,
      mesh=mesh,
      out_type=jax.ShapeDtypeStruct((n_rows, D_MODEL), jnp.float32),
      scratch_types=scratch,
      compiler_params=pltpu.CompilerParams(use_tc_tiling_on_sc=False),
  )
  def k(idx_hbm, table_hbm, out_hbm, idx_v, buf0, buf1, sem0, sem1):
    bufs = (buf0, buf1)
    sems = (sem0, sem1)
    wid = lax.axis_index("s") * NUM_CORES + lax.axis_index("c")
    base = wid * (n_chunks * CHUNK)

    pltpu.sync_copy(idx_hbm.at[wid], idx_v)

    for b in range(NBUF):
      pltpu.async_copy(table_hbm.at[idx_v.at[b]], bufs[b], sems[b])

    def chunk_step(c, b):
      buf = bufs[b]
      pltpu.make_async_copy(table_hbm.at[idx_v.at[c]], buf, sems[b]).wait()

      def row_body(r, carry):
        for rr in range(4):
          for kk in range(D_MODEL // 16):
            sl = (r * 4 + rr, pl.ds(kk * 16, 16))
            buf[sl] = buf[sl] * SCALE
        return carry
      lax.fori_loop(0, CHUNK // 4, row_body, 0, unroll=False)

      pltpu.sync_copy(buf, out_hbm.at[pl.ds(base + c * CHUNK, CHUNK)])

      nxt = c + NBUF

      @pl.when(nxt < n_chunks)
      def _():
        pltpu.async_copy(table_hbm.at[idx_v.at[nxt]], buf, sems[b])

    def outer(i, carry):
      for b in range(NBUF):
        chunk_step(i * NBUF + b, b)
      return carry

    lax.fori_loop(0, n_chunks // NBUF, outer, 0, unroll=False)

  return k(idx3, table)


def kernel(x, emb_table):
  batch, seq = x.shape
  total = batch * seq
  assert total % (NUM_WORKERS * CHUNK) == 0
  n_chunks = total // (NUM_WORKERS * CHUNK)
  idx3 = x.reshape(NUM_WORKERS, n_chunks, CHUNK).astype(jnp.int32)
  out = _emb_lookup(idx3, emb_table, n_chunks)
  return out.reshape(batch, seq, D_MODEL)


# speedup vs baseline: 3.6383x; 1.0431x over previous
import functools
import math

import jax
import jax.numpy as jnp
from jax import lax
from jax.experimental import pallas as pl
from jax.experimental.pallas import tpu as pltpu
from jax.experimental.pallas import tpu_sc as plsc

D_MODEL = 64
SCALE = math.sqrt(D_MODEL)

NUM_CORES = 2
NUM_SUBCORES = 16
NUM_WORKERS = NUM_CORES * NUM_SUBCORES
CHUNK = 128
NBUF = 5
ROWS_PER_STEP = 8


@functools.partial(jax.jit, static_argnums=(2,))
def _emb_lookup(idx3, table, n_chunks):
  assert n_chunks % NBUF == 0 and n_chunks // NBUF >= 2
  n_outer = n_chunks // NBUF
  mesh = plsc.VectorSubcoreMesh(core_axis_name="c", subcore_axis_name="s")
  n_rows = NUM_WORKERS * n_chunks * CHUNK

  scratch = [pltpu.VMEM((n_chunks, CHUNK), jnp.int32)]
  scratch += [pltpu.VMEM((CHUNK, D_MODEL), jnp.float32) for _ in range(2 * NBUF)]
  scratch += [pltpu.SemaphoreType.DMA for _ in range(2 * NBUF)]

  @functools.partial(
      pl.kernel,
      mesh=mesh,
      out_type=jax.ShapeDtypeStruct((n_rows, D_MODEL), jnp.float32),
      scratch_types=scratch,
      compiler_params=pltpu.CompilerParams(use_tc_tiling_on_sc=False),
  )
  def k(idx_hbm, table_hbm, out_hbm, idx_v, *bufs_and_sems):
    in_bufs = bufs_and_sems[:NBUF]
    out_bufs = bufs_and_sems[NBUF:2 * NBUF]
    g_sems = bufs_and_sems[2 * NBUF:3 * NBUF]
    s_sems = bufs_and_sems[3 * NBUF:4 * NBUF]
    wid = lax.axis_index("s") * NUM_CORES + lax.axis_index("c")
    base = wid * (n_chunks * CHUNK)

    pltpu.sync_copy(idx_hbm.at[wid], idx_v)

    def fire_gather(c, b):
      pltpu.async_copy(table_hbm.at[idx_v.at[c]], in_bufs[b], g_sems[b])

    def wait_gather(c, b):
      pltpu.make_async_copy(
          table_hbm.at[idx_v.at[c]], in_bufs[b], g_sems[b]).wait()

    def fire_scatter(c, b):
      pltpu.async_copy(
          out_bufs[b], out_hbm.at[pl.ds(base + c * CHUNK, CHUNK)], s_sems[b])

    def wait_scatter(c, b):
      pltpu.make_async_copy(
          out_bufs[b], out_hbm.at[pl.ds(base + c * CHUNK, CHUNK)],
          s_sems[b]).wait()

    def scale(b):
      src, dst = in_bufs[b], out_bufs[b]

      def body(r, carry):
        for rr in range(ROWS_PER_STEP):
          for kk in range(D_MODEL // 16):
            sl = (r * ROWS_PER_STEP + rr, pl.ds(kk * 16, 16))
            dst[sl] = src[sl] * SCALE
        return carry

      lax.fori_loop(0, CHUNK // ROWS_PER_STEP, body, 0, unroll=False)

    for b in range(NBUF):
      fire_gather(b, b)

    for b in range(NBUF):
      wait_gather(b, b)
      scale(b)
      fire_gather(NBUF + b, b)
      fire_scatter(b, b)

    def outer(i, carry):
      c0 = i * NBUF
      for b in range(NBUF):
        wait_gather(c0 + b, b)
        wait_scatter(c0 - NBUF + b, b)
        scale(b)
        fire_gather(c0 + NBUF + b, b)
        fire_scatter(c0 + b, b)
      return carry

    lax.fori_loop(1, n_outer - 1, outer, 0, unroll=False)

    c0 = (n_outer - 1) * NBUF
    for b in range(NBUF):
      wait_gather(c0 + b, b)
      wait_scatter(c0 - NBUF + b, b)
      scale(b)
      fire_scatter(c0 + b, b)

    for b in range(NBUF):
      wait_scatter(c0 + b, b)

  return k(idx3, table)


def kernel(x, emb_table):
  batch, seq = x.shape
  total = batch * seq
  assert total % (NUM_WORKERS * CHUNK) == 0
  n_chunks = total // (NUM_WORKERS * CHUNK)
  idx3 = x.reshape(NUM_WORKERS, n_chunks, CHUNK).astype(jnp.int32)
  out = _emb_lookup(idx3, emb_table, n_chunks)
  return out.reshape(batch, seq, D_MODEL)
